# Initial kernel scaffold; baseline (speedup 1.0000x reference)
import jax
import jax.numpy as jnp
from jax.experimental import pallas as pl


def kernel(x, W_pre, b_pre, temperature, W_dgm, b_dgm, W_conv, b_conv, W_fc, b_fc):
    raise NotImplementedError("write your pallas kernel here")


# speedup vs baseline: 4.7578x; 4.7578x over previous
import functools

import jax
import jax.numpy as jnp
from jax import lax
from jax.experimental import pallas as pl
from jax.experimental.pallas import tpu as pltpu

N = 4096
K_NN = 16
D_EMB = 64


def _k1_body(x_ref, wpre_ref, bpre_ref, wdgm_ref, bdgm_ref, wconv_ref,
             bconv_ref, gx_ref, sq_ref, pp_ref, q_ref):
    pre = jnp.dot(x_ref[...], wpre_ref[...],
                  preferred_element_type=jnp.float32) + bpre_ref[...]
    xp = jnp.where(pre >= 0, pre, 0.1 * pre)
    gx = jnp.dot(xp, wdgm_ref[...],
                 preferred_element_type=jnp.float32) + bdgm_ref[...]
    gx_ref[...] = gx
    sq_ref[...] = jnp.sum(gx * gx, axis=1, keepdims=True)
    wc = wconv_ref[...]
    w_top = wc[:D_EMB, :]
    w_bot = wc[D_EMB:, :]
    pp_ref[...] = jnp.dot(xp, w_top - w_bot,
                          preferred_element_type=jnp.float32) + bconv_ref[...]
    q_ref[...] = jnp.dot(xp, w_bot, preferred_element_type=jnp.float32)


def _k2_body(gxr_ref, gx_ref, sqt_ref, sqr_ref, t_ref, idx_ref):
    d = (sqr_ref[...] + sqt_ref[...] - 2.0 * lax.dot_general(
        gxr_ref[...], gx_ref[...], (((1,), (1,)), ((), ())),
        preferred_element_type=jnp.float32)) * t_ref[0, 0]
    rows = d.shape[0]
    iota = lax.broadcasted_iota(jnp.int32, (rows, N), 1)
    cols = []
    for _ in range(K_NN):
        v = jnp.min(d, axis=1, keepdims=True)
        am = jnp.min(jnp.where(d == v, iota, jnp.int32(N)), axis=1)
        cols.append(am)
        d = jnp.where(iota == am[:, None], jnp.inf, d)
    idx_ref[...] = jnp.stack(cols, axis=1)


def _k5_body(xo_ref, wfc_ref, bfc_ref, out_ref):
    out_ref[...] = jnp.dot(xo_ref[...], wfc_ref[...],
                           preferred_element_type=jnp.float32) + bfc_ref[...]


def kernel(x, W_pre, b_pre, temperature, W_dgm, b_dgm, W_conv, b_conv, W_fc,
           b_fc):
    x2 = x[0]

    gx, sq, pp, q = pl.pallas_call(
        _k1_body,
        out_shape=[
            jax.ShapeDtypeStruct((N, D_EMB), jnp.float32),
            jax.ShapeDtypeStruct((N, 1), jnp.float32),
            jax.ShapeDtypeStruct((N, D_EMB), jnp.float32),
            jax.ShapeDtypeStruct((N, D_EMB), jnp.float32),
        ],
    )(x2, W_pre, b_pre.reshape(1, -1), W_dgm, b_dgm.reshape(1, -1), W_conv,
      b_conv.reshape(1, -1))

    t = jnp.exp(jnp.clip(temperature, -5.0, 5.0))
    sqt = sq.reshape(1, N)
    rows = 256
    idx = pl.pallas_call(
        _k2_body,
        grid=(N // rows,),
        in_specs=[
            pl.BlockSpec((rows, D_EMB), lambda i: (i, 0)),
            pl.BlockSpec((N, D_EMB), lambda i: (0, 0)),
            pl.BlockSpec((1, N), lambda i: (0, 0)),
            pl.BlockSpec((rows, 1), lambda i: (i, 0)),
            pl.BlockSpec((1, 1), lambda i: (0, 0)),
        ],
        out_specs=pl.BlockSpec((rows, K_NN), lambda i: (i, 0)),
        out_shape=jax.ShapeDtypeStruct((N, K_NN), jnp.int32),
    )(gx, gx, sqt, sq, t.reshape(1, 1))

    m = jnp.arange(N * K_NN)
    a_idx = idx[m % N, m // N]
    b_idx = m // K_NN
    lp = (-t * jnp.sum((gx[a_idx] - gx[b_idx]) ** 2, axis=-1))

    mm = jnp.arange(N * K_NN // 2)
    src_e = idx[mm % N, mm // N]
    seg_e = idx[(mm + N * K_NN // 2) % N, (mm + N * K_NN // 2) // N]
    qq = jnp.arange(2048)
    src_all = jnp.concatenate([src_e, qq])
    seg_all = jnp.concatenate([seg_e, 2048 + qq])
    mseg = jax.ops.segment_max(q[src_all], seg_all, num_segments=N)
    xo = jnp.maximum(pp + mseg, 0.0)
    xo = jnp.where(jnp.isfinite(xo), xo, 0.0)

    out = pl.pallas_call(
        _k5_body,
        out_shape=jax.ShapeDtypeStruct((N, 32), jnp.float32),
    )(xo, W_fc, b_fc.reshape(1, -1))

    return out[None], lp.reshape(1, N, K_NN, 1)


# speedup vs baseline: 7.4819x; 1.5726x over previous
import jax
import jax.numpy as jnp
from jax import lax
from jax.experimental import pallas as pl
from jax.experimental.pallas import tpu as pltpu
from jax.experimental.pallas import tpu_sc as plsc

N = 4096
K_NN = 16
D_EMB = 64
DP = 128
NTILES = 32
SEGS_PER_TILE = N // NTILES
EDG = N * K_NN // 2
EPT = N * K_NN // NTILES


def _lanes(v):
    return jnp.full((16,), v, jnp.int32)


def _iota16():
    return lax.broadcasted_iota(jnp.int32, (16,), 0)


def _k1_body(x_ref, wpre_ref, bpre_ref, wdgm_ref, bdgm_ref, wconv_ref,
             bconv_ref, gx_ref, gx128_ref, sq_ref, pp_ref, q128_ref):
    pre = jnp.dot(x_ref[...], wpre_ref[...],
                  preferred_element_type=jnp.float32) + bpre_ref[...]
    xp = jnp.where(pre >= 0, pre, 0.1 * pre)
    gx = jnp.dot(xp, wdgm_ref[...],
                 preferred_element_type=jnp.float32) + bdgm_ref[...]
    gx_ref[...] = gx
    zpad = jnp.zeros((N, DP - D_EMB), jnp.float32)
    gx128_ref[...] = jnp.concatenate([gx, zpad], axis=1)
    sq_ref[...] = jnp.sum(gx * gx, axis=1, keepdims=True)
    wc = wconv_ref[...]
    w_top = wc[:D_EMB, :]
    w_bot = wc[D_EMB:, :]
    pp_ref[...] = jnp.dot(xp, w_top - w_bot,
                          preferred_element_type=jnp.float32) + bconv_ref[...]
    q = jnp.dot(xp, w_bot, preferred_element_type=jnp.float32)
    q128_ref[...] = jnp.concatenate([q, zpad], axis=1)


def _k2_body(gxr_ref, gx_ref, sqt_ref, sqr_ref, t_ref, idx_ref):
    d = (sqr_ref[...] + sqt_ref[...] - 2.0 * lax.dot_general(
        gxr_ref[...], gx_ref[...], (((1,), (1,)), ((), ())),
        preferred_element_type=jnp.float32)) * t_ref[0, 0]
    rows = d.shape[0]
    iota = lax.broadcasted_iota(jnp.int32, (rows, N), 1)
    cols = []
    for _ in range(K_NN):
        v = jnp.min(d, axis=1, keepdims=True)
        am = jnp.min(jnp.where(d == v, iota, jnp.int32(N)), axis=1)
        cols.append(am)
        d = jnp.where(iota == am[:, None], jnp.inf, d)
    idx_ref[...] = jnp.stack(cols, axis=1)


def _sc_wid():
    return lax.axis_index("s") * 2 + lax.axis_index("c")


def _k4_body(idxt_hbm, gx128_hbm, g1_hbm, alist_v, arows_v, sem):
    w = _sc_wid()
    pltpu.sync_copy(idxt_hbm.at[pl.ds(w * EPT, EPT)], alist_v)

    def chunk(cc, _):
        pltpu.async_copy(gx128_hbm.at[alist_v.at[pl.ds(cc * 512, 512)]],
                         arows_v, sem).wait()
        pltpu.sync_copy(arows_v, g1_hbm.at[pl.ds(w * EPT + cc * 512, 512)])
        return 0

    lax.fori_loop(0, EPT // 512, chunk, 0)


def _k4b_body(g1_ref, gxr_ref, t_ref, lp_ref):
    diff = g1_ref[...][:, :, :D_EMB] - gxr_ref[...][:, None, :]
    lp_ref[...] = -t_ref[0, 0] * jnp.sum(diff * diff, axis=2)


def _k3_body(idxt_hbm, q128_hbm, ppf_hbm, out_hbm, seg_st, src_st, pk_own,
             m_v, qrows_v, glist_v, slist_v, prows_v, xov_v, sem):
    w = _sc_wid()
    lo = w * SEGS_PER_TILE
    it = _iota16()
    neg = jnp.full((16,), -jnp.inf, jnp.float32)

    def minit(i, _):
        m_v[pl.ds(i * 16, 16)] = neg
        return 0

    lax.fori_loop(0, SEGS_PER_TILE * D_EMB // 16, minit, 0, unroll=8)

    def scan_chunk(ch, total):
        pltpu.sync_copy(idxt_hbm.at[pl.ds(EDG + ch * 4096, 4096)], seg_st)
        pltpu.sync_copy(idxt_hbm.at[pl.ds(ch * 4096, 4096)], src_st)

        def scan_vec(i, tot):
            sv = seg_st[pl.ds(i * 16, 16)]
            ov = lax.shift_right_logical(sv, 7) ^ _lanes(w)
            s8 = [jnp.minimum(ov[2 * a], ov[2 * a + 1]) for a in range(8)]
            s4 = [jnp.minimum(s8[2 * a], s8[2 * a + 1]) for a in range(4)]
            s2 = [jnp.minimum(s4[0], s4[1]), jnp.minimum(s4[2], s4[3])]
            anyown = jnp.minimum(s2[0], s2[1]) == 0

            def append_lanes():
                xv = src_st[pl.ds(i * 16, 16)]
                pk = jnp.bitwise_or(lax.shift_left(sv, 12), xv)
                tot2 = tot
                for ji in range(16):
                    owned = ov[ji] == 0

                    @pl.when(owned)
                    def _(ji=ji, tot2=tot2):
                        base = tot2 & ~jnp.int32(15)
                        lane = tot2 & jnp.int32(15)
                        cur = pk_own[pl.ds(base, 16)]
                        pk_own[pl.ds(base, 16)] = jnp.where(
                            it == lane, pk[ji], cur)

                    tot2 = tot2 + jnp.where(owned, 1, 0)
                return tot2

            return lax.cond(anyown, append_lanes, lambda: tot)

        return lax.fori_loop(0, 256, scan_vec, total)

    total = lax.fori_loop(0, EDG // 4096, scan_chunk, jnp.int32(0))

    zz = jnp.zeros((16,), jnp.int32)
    for j in range(8):
        pk_own[pl.ds(total + j * 16, 16)] = zz

    @pl.when(w >= 16)
    def _():
        q0 = (w - 16) * SEGS_PER_TILE
        pltpu.sync_copy(q128_hbm.at[pl.ds(q0, SEGS_PER_TILE)], qrows_v)

        def omax(jg, _2):
            for ji in range(16):
                s = jg * 16 + ji
                for c4 in range(4):
                    off = s * D_EMB + c4 * 16
                    m_v[pl.ds(off, 16)] = jnp.maximum(
                        m_v[pl.ds(off, 16)], qrows_v[s, pl.ds(c4 * 16, 16)])
            return 0

        lax.fori_loop(0, SEGS_PER_TILE // 16, omax, 0)

    nch = lax.div(total + 127, 128)

    def echunk(ci, _):
        base = ci * 128
        for k16 in range(8):
            pk = pk_own[pl.ds(base + k16 * 16, 16)]
            glist_v[pl.ds(k16 * 16, 16)] = jnp.bitwise_and(pk, 4095)
            slist_v[pl.ds(k16 * 16, 16)] = (
                lax.shift_right_logical(pk, 12) - _lanes(lo))
        pltpu.async_copy(q128_hbm.at[glist_v], qrows_v, sem).wait()

        def erow(jg, _2):
            segv = slist_v[pl.ds(jg * 16, 16)]
            for ji in range(16):
                j = jg * 16 + ji

                @pl.when(base + j < total)
                def _3(j=j, ji=ji, segv=segv):
                    off = segv[ji] * D_EMB
                    for c4 in range(4):
                        m_v[pl.ds(off + c4 * 16, 16)] = jnp.maximum(
                            m_v[pl.ds(off + c4 * 16, 16)],
                            qrows_v[j, pl.ds(c4 * 16, 16)])
            return 0

        return lax.fori_loop(0, 8, erow, 0)

    lax.fori_loop(0, nch, echunk, 0)

    pltpu.sync_copy(ppf_hbm.at[pl.ds(lo * D_EMB, SEGS_PER_TILE * D_EMB)],
                    prows_v)

    def fin(i, _):
        xov_v[pl.ds(i * 16, 16)] = jnp.maximum(
            prows_v[pl.ds(i * 16, 16)] + m_v[pl.ds(i * 16, 16)], 0.0)
        return 0

    lax.fori_loop(0, SEGS_PER_TILE * D_EMB // 16, fin, 0, unroll=8)
    pltpu.sync_copy(xov_v, out_hbm.at[pl.ds(lo * D_EMB,
                                            SEGS_PER_TILE * D_EMB)])


def _k5_body(xo_ref, wfc_ref, bfc_ref, out_ref):
    out_ref[...] = jnp.dot(xo_ref[...], wfc_ref[...],
                           preferred_element_type=jnp.float32) + bfc_ref[...]


def kernel(x, W_pre, b_pre, temperature, W_dgm, b_dgm, W_conv, b_conv, W_fc,
           b_fc):
    x2 = x[0]

    gx, gx128, sq, pp, q128 = pl.pallas_call(
        _k1_body,
        out_shape=[
            jax.ShapeDtypeStruct((N, D_EMB), jnp.float32),
            jax.ShapeDtypeStruct((N, DP), jnp.float32),
            jax.ShapeDtypeStruct((N, 1), jnp.float32),
            jax.ShapeDtypeStruct((N, D_EMB), jnp.float32),
            jax.ShapeDtypeStruct((N, DP), jnp.float32),
        ],
    )(x2, W_pre, b_pre.reshape(1, -1), W_dgm, b_dgm.reshape(1, -1), W_conv,
      b_conv.reshape(1, -1))

    t = jnp.exp(jnp.clip(temperature, -5.0, 5.0))
    sqt = sq.reshape(1, N)
    rows = 256
    idx = pl.pallas_call(
        _k2_body,
        grid=(N // rows,),
        in_specs=[
            pl.BlockSpec((rows, D_EMB), lambda i: (i, 0)),
            pl.BlockSpec((N, D_EMB), lambda i: (0, 0)),
            pl.BlockSpec((1, N), lambda i: (0, 0)),
            pl.BlockSpec((rows, 1), lambda i: (i, 0)),
            pl.BlockSpec((1, 1), lambda i: (0, 0)),
        ],
        out_specs=pl.BlockSpec((rows, K_NN), lambda i: (i, 0)),
        out_shape=jax.ShapeDtypeStruct((N, K_NN), jnp.int32),
    )(gx, gx, sqt, sq, t.reshape(1, 1))

    mesh = plsc.VectorSubcoreMesh(core_axis_name="c", subcore_axis_name="s")
    idx_t = idx.T.reshape(-1)

    g1 = pl.kernel(
        _k4_body,
        out_type=jax.ShapeDtypeStruct((N * K_NN, DP), jnp.float32),
        mesh=mesh,
        scratch_types=[
            pltpu.VMEM((EPT,), jnp.int32),
            pltpu.VMEM((512, DP), jnp.float32),
            pltpu.SemaphoreType.DMA,
        ],
    )(idx_t, gx128)

    lp = pl.pallas_call(
        _k4b_body,
        grid=(8,),
        in_specs=[
            pl.BlockSpec((512, K_NN, DP), lambda i: (i, 0, 0)),
            pl.BlockSpec((512, D_EMB), lambda i: (i, 0)),
            pl.BlockSpec((1, 1), lambda i: (0, 0)),
        ],
        out_specs=pl.BlockSpec((512, K_NN), lambda i: (i, 0)),
        out_shape=jax.ShapeDtypeStruct((N, K_NN), jnp.float32),
    )(g1.reshape(N, K_NN, DP), gx, t.reshape(1, 1))

    xo_f = pl.kernel(
        _k3_body,
        out_type=jax.ShapeDtypeStruct((N * D_EMB,), jnp.float32),
        mesh=mesh,
        scratch_types=[
            pltpu.VMEM((4096,), jnp.int32),
            pltpu.VMEM((4096,), jnp.int32),
            pltpu.VMEM((EDG + 256,), jnp.int32),
            pltpu.VMEM((SEGS_PER_TILE * D_EMB,), jnp.float32),
            pltpu.VMEM((128, DP), jnp.float32),
            pltpu.VMEM((128,), jnp.int32),
            pltpu.VMEM((128,), jnp.int32),
            pltpu.VMEM((SEGS_PER_TILE * D_EMB,), jnp.float32),
            pltpu.VMEM((SEGS_PER_TILE * D_EMB,), jnp.float32),
            pltpu.SemaphoreType.DMA,
        ],
    )(idx_t, q128, pp.reshape(-1))
    xo = xo_f.reshape(N, D_EMB)

    out = pl.pallas_call(
        _k5_body,
        out_shape=jax.ShapeDtypeStruct((N, 32), jnp.float32),
    )(xo, W_fc, b_fc.reshape(1, -1))

    return out[None], lp.reshape(1, N, K_NN, 1)


# speedup vs baseline: 8.0051x; 1.0699x over previous
import jax
import jax.numpy as jnp
from jax import lax
from jax.experimental import pallas as pl
from jax.experimental.pallas import tpu as pltpu
from jax.experimental.pallas import tpu_sc as plsc

N = 4096
K_NN = 16
D_EMB = 64
DP = 128
NTILES = 32
SEGS_PER_TILE = N // NTILES
EDG = N * K_NN // 2
EPT = N * K_NN // NTILES


def _lanes(v):
    return jnp.full((16,), v, jnp.int32)


def _iota16():
    return lax.broadcasted_iota(jnp.int32, (16,), 0)


def _k1_body(x_ref, wpre_ref, bpre_ref, wdgm_ref, bdgm_ref, wconv_ref,
             bconv_ref, gx_ref, gx128_ref, sq_ref, pp_ref, q128_ref):
    pre = jnp.dot(x_ref[...], wpre_ref[...],
                  preferred_element_type=jnp.float32) + bpre_ref[...]
    xp = jnp.where(pre >= 0, pre, 0.1 * pre)
    gx = jnp.dot(xp, wdgm_ref[...],
                 preferred_element_type=jnp.float32) + bdgm_ref[...]
    gx_ref[...] = gx
    zpad = jnp.zeros((N, DP - D_EMB), jnp.float32)
    gx128_ref[...] = jnp.concatenate([gx, zpad], axis=1)
    sq_ref[...] = jnp.sum(gx * gx, axis=1, keepdims=True)
    wc = wconv_ref[...]
    w_top = wc[:D_EMB, :]
    w_bot = wc[D_EMB:, :]
    pp_ref[...] = jnp.dot(xp, w_top - w_bot,
                          preferred_element_type=jnp.float32) + bconv_ref[...]
    q = jnp.dot(xp, w_bot, preferred_element_type=jnp.float32)
    q128_ref[...] = jnp.concatenate([q, zpad], axis=1)


def _k2_body(gxr_ref, gx_ref, sqt_ref, sqr_ref, t_ref, idx_ref):
    d = (sqr_ref[...] + sqt_ref[...] - 2.0 * lax.dot_general(
        gxr_ref[...], gx_ref[...], (((1,), (1,)), ((), ())),
        preferred_element_type=jnp.float32)) * t_ref[0, 0]
    rows = d.shape[0]
    iota = lax.broadcasted_iota(jnp.int32, (rows, N), 1)
    cols = []
    for _ in range(K_NN):
        am = jnp.argmin(d, axis=1).astype(jnp.int32)
        cols.append(am)
        d = jnp.where(iota == am[:, None], jnp.inf, d)
    idx_ref[...] = jnp.stack(cols, axis=1)


def _sc_wid():
    return lax.axis_index("s") * 2 + lax.axis_index("c")


def _k4_body(idxt_hbm, gx128_hbm, g1_hbm, alist_v, arows_v, sem):
    w = _sc_wid()
    pltpu.sync_copy(idxt_hbm.at[pl.ds(w * EPT, EPT)], alist_v)

    def chunk(cc, _):
        pltpu.async_copy(gx128_hbm.at[alist_v.at[pl.ds(cc * 512, 512)]],
                         arows_v, sem).wait()
        pltpu.sync_copy(arows_v, g1_hbm.at[pl.ds(w * EPT + cc * 512, 512)])
        return 0

    lax.fori_loop(0, EPT // 512, chunk, 0)


def _k4b_body(g1_ref, gxr_ref, t_ref, lp_ref):
    diff = g1_ref[...][:, :, :D_EMB] - gxr_ref[...][:, None, :]
    lp_ref[...] = -t_ref[0, 0] * jnp.sum(diff * diff, axis=2)


def _k3_body(idxt_hbm, q128_hbm, ppf_hbm, out_hbm, seg_st, src_st, pk_own,
             m_v, qrows_v, glist_v, slist_v, prows_v, xov_v, sem):
    w = _sc_wid()
    lo = w * SEGS_PER_TILE
    it = _iota16()
    neg = jnp.full((16,), -jnp.inf, jnp.float32)

    def minit(i, _):
        m_v[pl.ds(i * 16, 16)] = neg
        return 0

    lax.fori_loop(0, SEGS_PER_TILE * D_EMB // 16, minit, 0, unroll=8)

    def scan_chunk(ch, total):
        pltpu.sync_copy(idxt_hbm.at[pl.ds(EDG + ch * 4096, 4096)], seg_st)
        pltpu.sync_copy(idxt_hbm.at[pl.ds(ch * 4096, 4096)], src_st)

        def scan_vec(i, tot):
            sv = seg_st[pl.ds(i * 16, 16)]
            ov = lax.shift_right_logical(sv, 7) ^ _lanes(w)
            s8 = [jnp.minimum(ov[2 * a], ov[2 * a + 1]) for a in range(8)]
            s4 = [jnp.minimum(s8[2 * a], s8[2 * a + 1]) for a in range(4)]
            s2 = [jnp.minimum(s4[0], s4[1]), jnp.minimum(s4[2], s4[3])]
            anyown = jnp.minimum(s2[0], s2[1]) == 0

            def append_lanes():
                xv = src_st[pl.ds(i * 16, 16)]
                pk = jnp.bitwise_or(lax.shift_left(sv, 12), xv)
                tot2 = tot
                for ji in range(16):
                    owned = ov[ji] == 0

                    @pl.when(owned)
                    def _(ji=ji, tot2=tot2):
                        base = tot2 & ~jnp.int32(15)
                        lane = tot2 & jnp.int32(15)
                        cur = pk_own[pl.ds(base, 16)]
                        pk_own[pl.ds(base, 16)] = jnp.where(
                            it == lane, pk[ji], cur)

                    tot2 = tot2 + jnp.where(owned, 1, 0)
                return tot2

            return lax.cond(anyown, append_lanes, lambda: tot)

        return lax.fori_loop(0, 256, scan_vec, total)

    total = lax.fori_loop(0, EDG // 4096, scan_chunk, jnp.int32(0))

    zz = jnp.zeros((16,), jnp.int32)
    for j in range(8):
        pk_own[pl.ds(total + j * 16, 16)] = zz

    @pl.when(w >= 16)
    def _():
        q0 = (w - 16) * SEGS_PER_TILE
        pltpu.sync_copy(q128_hbm.at[pl.ds(q0, SEGS_PER_TILE)], qrows_v)

        def omax(jg, _2):
            for ji in range(16):
                s = jg * 16 + ji
                for c4 in range(4):
                    off = s * D_EMB + c4 * 16
                    m_v[pl.ds(off, 16)] = jnp.maximum(
                        m_v[pl.ds(off, 16)], qrows_v[s, pl.ds(c4 * 16, 16)])
            return 0

        lax.fori_loop(0, SEGS_PER_TILE // 16, omax, 0)

    nch = lax.div(total + 127, 128)

    def echunk(ci, _):
        base = ci * 128
        for k16 in range(8):
            pk = pk_own[pl.ds(base + k16 * 16, 16)]
            glist_v[pl.ds(k16 * 16, 16)] = jnp.bitwise_and(pk, 4095)
            slist_v[pl.ds(k16 * 16, 16)] = (
                lax.shift_right_logical(pk, 12) - _lanes(lo))
        pltpu.async_copy(q128_hbm.at[glist_v], qrows_v, sem).wait()

        def erow(jg, _2):
            segv = slist_v[pl.ds(jg * 16, 16)]
            for ji in range(16):
                j = jg * 16 + ji

                @pl.when(base + j < total)
                def _3(j=j, ji=ji, segv=segv):
                    off = segv[ji] * D_EMB
                    for c4 in range(4):
                        m_v[pl.ds(off + c4 * 16, 16)] = jnp.maximum(
                            m_v[pl.ds(off + c4 * 16, 16)],
                            qrows_v[j, pl.ds(c4 * 16, 16)])
            return 0

        return lax.fori_loop(0, 8, erow, 0)

    lax.fori_loop(0, nch, echunk, 0)

    pltpu.sync_copy(ppf_hbm.at[pl.ds(lo * D_EMB, SEGS_PER_TILE * D_EMB)],
                    prows_v)

    def fin(i, _):
        xov_v[pl.ds(i * 16, 16)] = jnp.maximum(
            prows_v[pl.ds(i * 16, 16)] + m_v[pl.ds(i * 16, 16)], 0.0)
        return 0

    lax.fori_loop(0, SEGS_PER_TILE * D_EMB // 16, fin, 0, unroll=8)
    pltpu.sync_copy(xov_v, out_hbm.at[pl.ds(lo * D_EMB,
                                            SEGS_PER_TILE * D_EMB)])


def _k5_body(xo_ref, wfc_ref, bfc_ref, out_ref):
    out_ref[...] = jnp.dot(xo_ref[...], wfc_ref[...],
                           preferred_element_type=jnp.float32) + bfc_ref[...]


def kernel(x, W_pre, b_pre, temperature, W_dgm, b_dgm, W_conv, b_conv, W_fc,
           b_fc):
    x2 = x[0]

    gx, gx128, sq, pp, q128 = pl.pallas_call(
        _k1_body,
        out_shape=[
            jax.ShapeDtypeStruct((N, D_EMB), jnp.float32),
            jax.ShapeDtypeStruct((N, DP), jnp.float32),
            jax.ShapeDtypeStruct((N, 1), jnp.float32),
            jax.ShapeDtypeStruct((N, D_EMB), jnp.float32),
            jax.ShapeDtypeStruct((N, DP), jnp.float32),
        ],
    )(x2, W_pre, b_pre.reshape(1, -1), W_dgm, b_dgm.reshape(1, -1), W_conv,
      b_conv.reshape(1, -1))

    t = jnp.exp(jnp.clip(temperature, -5.0, 5.0))
    sqt = sq.reshape(1, N)
    rows = 256
    idx = pl.pallas_call(
        _k2_body,
        grid=(N // rows,),
        in_specs=[
            pl.BlockSpec((rows, D_EMB), lambda i: (i, 0)),
            pl.BlockSpec((N, D_EMB), lambda i: (0, 0)),
            pl.BlockSpec((1, N), lambda i: (0, 0)),
            pl.BlockSpec((rows, 1), lambda i: (i, 0)),
            pl.BlockSpec((1, 1), lambda i: (0, 0)),
        ],
        out_specs=pl.BlockSpec((rows, K_NN), lambda i: (i, 0)),
        out_shape=jax.ShapeDtypeStruct((N, K_NN), jnp.int32),
    )(gx, gx, sqt, sq, t.reshape(1, 1))

    mesh = plsc.VectorSubcoreMesh(core_axis_name="c", subcore_axis_name="s")
    idx_t = idx.T.reshape(-1)

    g1 = pl.kernel(
        _k4_body,
        out_type=jax.ShapeDtypeStruct((N * K_NN, DP), jnp.float32),
        mesh=mesh,
        scratch_types=[
            pltpu.VMEM((EPT,), jnp.int32),
            pltpu.VMEM((512, DP), jnp.float32),
            pltpu.SemaphoreType.DMA,
        ],
    )(idx_t, gx128)

    lp = pl.pallas_call(
        _k4b_body,
        grid=(8,),
        in_specs=[
            pl.BlockSpec((512, K_NN, DP), lambda i: (i, 0, 0)),
            pl.BlockSpec((512, D_EMB), lambda i: (i, 0)),
            pl.BlockSpec((1, 1), lambda i: (0, 0)),
        ],
        out_specs=pl.BlockSpec((512, K_NN), lambda i: (i, 0)),
        out_shape=jax.ShapeDtypeStruct((N, K_NN), jnp.float32),
    )(g1.reshape(N, K_NN, DP), gx, t.reshape(1, 1))

    xo_f = pl.kernel(
        _k3_body,
        out_type=jax.ShapeDtypeStruct((N * D_EMB,), jnp.float32),
        mesh=mesh,
        scratch_types=[
            pltpu.VMEM((4096,), jnp.int32),
            pltpu.VMEM((4096,), jnp.int32),
            pltpu.VMEM((EDG + 256,), jnp.int32),
            pltpu.VMEM((SEGS_PER_TILE * D_EMB,), jnp.float32),
            pltpu.VMEM((128, DP), jnp.float32),
            pltpu.VMEM((128,), jnp.int32),
            pltpu.VMEM((128,), jnp.int32),
            pltpu.VMEM((SEGS_PER_TILE * D_EMB,), jnp.float32),
            pltpu.VMEM((SEGS_PER_TILE * D_EMB,), jnp.float32),
            pltpu.SemaphoreType.DMA,
        ],
    )(idx_t, q128, pp.reshape(-1))
    xo = xo_f.reshape(N, D_EMB)

    out = pl.pallas_call(
        _k5_body,
        out_shape=jax.ShapeDtypeStruct((N, 32), jnp.float32),
    )(xo, W_fc, b_fc.reshape(1, -1))

    return out[None], lp.reshape(1, N, K_NN, 1)


# speedup vs baseline: 9.5955x; 1.1987x over previous
import jax
import jax.numpy as jnp
from jax import lax
from jax.experimental import pallas as pl
from jax.experimental.pallas import tpu as pltpu
from jax.experimental.pallas import tpu_sc as plsc

N = 4096
K_NN = 16
D_EMB = 64
DP = 128
NTILES = 32
SEGS_PER_TILE = N // NTILES
EDG = N * K_NN // 2
EPT = N * K_NN // NTILES


def _lanes(v):
    return jnp.full((16,), v, jnp.int32)


def _iota16():
    return lax.broadcasted_iota(jnp.int32, (16,), 0)


def _k1_body(x_ref, wpre_ref, bpre_ref, wdgm_ref, bdgm_ref, wconv_ref,
             bconv_ref, gx_ref, gx128_ref, sq_ref, pp_ref, q128_ref):
    pre = jnp.dot(x_ref[...], wpre_ref[...],
                  preferred_element_type=jnp.float32) + bpre_ref[...]
    xp = jnp.where(pre >= 0, pre, 0.1 * pre)
    gx = jnp.dot(xp, wdgm_ref[...],
                 preferred_element_type=jnp.float32) + bdgm_ref[...]
    gx_ref[...] = gx
    zpad = jnp.zeros((N, DP - D_EMB), jnp.float32)
    gx128_ref[...] = jnp.concatenate([gx, zpad], axis=1)
    sq_ref[...] = jnp.sum(gx * gx, axis=1, keepdims=True)
    wc = wconv_ref[...]
    w_top = wc[:D_EMB, :]
    w_bot = wc[D_EMB:, :]
    pp_ref[...] = jnp.dot(xp, w_top - w_bot,
                          preferred_element_type=jnp.float32) + bconv_ref[...]
    q = jnp.dot(xp, w_bot, preferred_element_type=jnp.float32)
    q128_ref[...] = jnp.concatenate([q, zpad], axis=1)


def _k2_body(gxr_ref, gx_ref, sqt_ref, sqr_ref, t_ref, idx_ref):
    d = (sqr_ref[...] + sqt_ref[...] - 2.0 * lax.dot_general(
        gxr_ref[...], gx_ref[...], (((1,), (1,)), ((), ())),
        preferred_element_type=jnp.float32)) * t_ref[0, 0]
    rows = d.shape[0]
    iota = lax.broadcasted_iota(jnp.int32, (rows, N), 1)
    cols = []
    for _ in range(K_NN):
        am = jnp.argmin(d, axis=1).astype(jnp.int32)
        cols.append(am)
        d = jnp.where(iota == am[:, None], jnp.inf, d)
    idx_ref[...] = jnp.stack(cols, axis=1)


def _sc_wid():
    return lax.axis_index("s") * 2 + lax.axis_index("c")


def _k4_body(idxt_hbm, gx128_hbm, g1_hbm, alist_v, arows_v, sem):
    w = _sc_wid()
    pltpu.sync_copy(idxt_hbm.at[pl.ds(w * EPT, EPT)], alist_v)

    def chunk(cc, _):
        pltpu.async_copy(gx128_hbm.at[alist_v.at[pl.ds(cc * 512, 512)]],
                         arows_v, sem).wait()
        pltpu.sync_copy(arows_v, g1_hbm.at[pl.ds(w * EPT + cc * 512, 512)])
        return 0

    lax.fori_loop(0, EPT // 512, chunk, 0)


def _k4b_body(g1_ref, gxr_ref, t_ref, lp_ref):
    diff = g1_ref[...][:, :, :D_EMB] - gxr_ref[...][:, None, :]
    lp_ref[...] = -t_ref[0, 0] * jnp.sum(diff * diff, axis=2)


BROW = 1152
EPS = EDG // NTILES


def _k3a_body(idxt_hbm, bkt_hbm, seg_st, src_st, bkt_v, cnt_v):
    w = _sc_wid()
    it = _iota16()
    pltpu.sync_copy(idxt_hbm.at[pl.ds(EDG + w * EPS, EPS)], seg_st)
    pltpu.sync_copy(idxt_hbm.at[pl.ds(w * EPS, EPS)], src_st)
    zz = jnp.zeros((16,), jnp.int32)

    def binit(i, _):
        bkt_v[pl.ds(i * 16, 16)] = zz
        return 0

    lax.fori_loop(0, NTILES * BROW // 16 + 1, binit, 0, unroll=8)
    cnt_v[pl.ds(0, 16)] = zz
    cnt_v[pl.ds(16, 16)] = zz
    cnt_v[pl.ds(32, 16)] = zz

    def scan_vec(i, _):
        sv = seg_st[pl.ds(i * 16, 16)]
        xv = src_st[pl.ds(i * 16, 16)]
        pk = jnp.bitwise_or(lax.shift_left(sv, 12), xv)
        ow = lax.shift_right_logical(sv, 7)
        for ji in range(16):
            o = ow[ji]
            cv = cnt_v[pl.ds(o, 16)]
            cnt = cv[0]
            addr = o * BROW + 1 + cnt
            base = addr & ~jnp.int32(15)
            lane = addr & jnp.int32(15)
            cur = bkt_v[pl.ds(base, 16)]
            bkt_v[pl.ds(base, 16)] = jnp.where(it == lane, pk[ji], cur)
            cnt_v[pl.ds(o, 16)] = jnp.where(it == 0, cnt + 1, cv)
        return 0

    lax.fori_loop(0, EPS // 16, scan_vec, 0)

    for o in range(NTILES):
        cv = cnt_v[pl.ds(o, 16)]
        head = bkt_v[pl.ds(o * BROW, 16)]
        bkt_v[pl.ds(o * BROW, 16)] = jnp.where(it == 0, cv[0], head)
        pltpu.sync_copy(bkt_v.at[pl.ds(o * BROW, BROW)],
                        bkt_hbm.at[o, w])


def _k3_body(q128_hbm, bkt_hbm, ppf_hbm, out_hbm, bkts_v, pk_own,
             m_v, qrows_v, glist_v, slist_v, prows_v, xov_v, sem):
    w = _sc_wid()
    lo = w * SEGS_PER_TILE
    it = _iota16()
    neg = jnp.full((16,), -jnp.inf, jnp.float32)

    def minit(i, _):
        m_v[pl.ds(i * 16, 16)] = neg
        return 0

    lax.fori_loop(0, SEGS_PER_TILE * D_EMB // 16, minit, 0, unroll=8)

    pltpu.sync_copy(bkt_hbm.at[w], bkts_v)
    total = jnp.int32(0)
    for s in range(NTILES):
        cnt_s = bkts_v[pl.ds(s * BROW, 16)][0]

        def copy_vec(kk, tot, s=s):
            pk_own[pl.ds(tot, 16)] = bkts_v[pl.ds(s * BROW + 1 + kk * 16,
                                                  16)]
            return tot + 16

        lax.fori_loop(0, lax.div(cnt_s + 15, 16), copy_vec, total)
        total = total + cnt_s

    zz = jnp.zeros((16,), jnp.int32)
    for j in range(8):
        pk_own[pl.ds(total + j * 16, 16)] = zz

    @pl.when(w >= 16)
    def _():
        q0 = (w - 16) * SEGS_PER_TILE
        pltpu.sync_copy(q128_hbm.at[pl.ds(q0, SEGS_PER_TILE)], qrows_v)

        def omax(jg, _2):
            for ji in range(16):
                s = jg * 16 + ji
                for c4 in range(4):
                    off = s * D_EMB + c4 * 16
                    m_v[pl.ds(off, 16)] = jnp.maximum(
                        m_v[pl.ds(off, 16)], qrows_v[s, pl.ds(c4 * 16, 16)])
            return 0

        lax.fori_loop(0, SEGS_PER_TILE // 16, omax, 0)

    nch = lax.div(total + 127, 128)

    def echunk(ci, _):
        base = ci * 128
        for k16 in range(8):
            pk = pk_own[pl.ds(base + k16 * 16, 16)]
            glist_v[pl.ds(k16 * 16, 16)] = jnp.bitwise_and(pk, 4095)
            slist_v[pl.ds(k16 * 16, 16)] = (
                lax.shift_right_logical(pk, 12) - _lanes(lo))
        pltpu.async_copy(q128_hbm.at[glist_v], qrows_v, sem).wait()

        def erow(jg, _2):
            segv = slist_v[pl.ds(jg * 16, 16)]
            for ji in range(16):
                j = jg * 16 + ji

                @pl.when(base + j < total)
                def _3(j=j, ji=ji, segv=segv):
                    off = segv[ji] * D_EMB
                    for c4 in range(4):
                        m_v[pl.ds(off + c4 * 16, 16)] = jnp.maximum(
                            m_v[pl.ds(off + c4 * 16, 16)],
                            qrows_v[j, pl.ds(c4 * 16, 16)])
            return 0

        return lax.fori_loop(0, 8, erow, 0)

    lax.fori_loop(0, nch, echunk, 0)

    pltpu.sync_copy(ppf_hbm.at[pl.ds(lo * D_EMB, SEGS_PER_TILE * D_EMB)],
                    prows_v)

    def fin(i, _):
        xov_v[pl.ds(i * 16, 16)] = jnp.maximum(
            prows_v[pl.ds(i * 16, 16)] + m_v[pl.ds(i * 16, 16)], 0.0)
        return 0

    lax.fori_loop(0, SEGS_PER_TILE * D_EMB // 16, fin, 0, unroll=8)
    pltpu.sync_copy(xov_v, out_hbm.at[pl.ds(lo * D_EMB,
                                            SEGS_PER_TILE * D_EMB)])


def _k5_body(xo_ref, wfc_ref, bfc_ref, out_ref):
    out_ref[...] = jnp.dot(xo_ref[...], wfc_ref[...],
                           preferred_element_type=jnp.float32) + bfc_ref[...]


def kernel(x, W_pre, b_pre, temperature, W_dgm, b_dgm, W_conv, b_conv, W_fc,
           b_fc):
    x2 = x[0]

    gx, gx128, sq, pp, q128 = pl.pallas_call(
        _k1_body,
        out_shape=[
            jax.ShapeDtypeStruct((N, D_EMB), jnp.float32),
            jax.ShapeDtypeStruct((N, DP), jnp.float32),
            jax.ShapeDtypeStruct((N, 1), jnp.float32),
            jax.ShapeDtypeStruct((N, D_EMB), jnp.float32),
            jax.ShapeDtypeStruct((N, DP), jnp.float32),
        ],
    )(x2, W_pre, b_pre.reshape(1, -1), W_dgm, b_dgm.reshape(1, -1), W_conv,
      b_conv.reshape(1, -1))

    t = jnp.exp(jnp.clip(temperature, -5.0, 5.0))
    sqt = sq.reshape(1, N)
    rows = 256
    idx = pl.pallas_call(
        _k2_body,
        grid=(N // rows,),
        in_specs=[
            pl.BlockSpec((rows, D_EMB), lambda i: (i, 0)),
            pl.BlockSpec((N, D_EMB), lambda i: (0, 0)),
            pl.BlockSpec((1, N), lambda i: (0, 0)),
            pl.BlockSpec((rows, 1), lambda i: (i, 0)),
            pl.BlockSpec((1, 1), lambda i: (0, 0)),
        ],
        out_specs=pl.BlockSpec((rows, K_NN), lambda i: (i, 0)),
        out_shape=jax.ShapeDtypeStruct((N, K_NN), jnp.int32),
    )(gx, gx, sqt, sq, t.reshape(1, 1))

    mesh = plsc.VectorSubcoreMesh(core_axis_name="c", subcore_axis_name="s")
    idx_t = idx.T.reshape(-1)

    g1 = pl.kernel(
        _k4_body,
        out_type=jax.ShapeDtypeStruct((N * K_NN, DP), jnp.float32),
        mesh=mesh,
        scratch_types=[
            pltpu.VMEM((EPT,), jnp.int32),
            pltpu.VMEM((512, DP), jnp.float32),
            pltpu.SemaphoreType.DMA,
        ],
    )(idx_t, gx128)

    lp = pl.pallas_call(
        _k4b_body,
        grid=(8,),
        in_specs=[
            pl.BlockSpec((512, K_NN, DP), lambda i: (i, 0, 0)),
            pl.BlockSpec((512, D_EMB), lambda i: (i, 0)),
            pl.BlockSpec((1, 1), lambda i: (0, 0)),
        ],
        out_specs=pl.BlockSpec((512, K_NN), lambda i: (i, 0)),
        out_shape=jax.ShapeDtypeStruct((N, K_NN), jnp.float32),
    )(g1.reshape(N, K_NN, DP), gx, t.reshape(1, 1))

    bkt = pl.kernel(
        _k3a_body,
        out_type=jax.ShapeDtypeStruct((NTILES, NTILES, BROW), jnp.int32),
        mesh=mesh,
        scratch_types=[
            pltpu.VMEM((EPS,), jnp.int32),
            pltpu.VMEM((EPS,), jnp.int32),
            pltpu.VMEM((NTILES * BROW + 16,), jnp.int32),
            pltpu.VMEM((48,), jnp.int32),
        ],
    )(idx_t)

    xo_f = pl.kernel(
        _k3_body,
        out_type=jax.ShapeDtypeStruct((N * D_EMB,), jnp.float32),
        mesh=mesh,
        scratch_types=[
            pltpu.VMEM((NTILES * BROW,), jnp.int32),
            pltpu.VMEM((EDG + 256,), jnp.int32),
            pltpu.VMEM((SEGS_PER_TILE * D_EMB,), jnp.float32),
            pltpu.VMEM((128, DP), jnp.float32),
            pltpu.VMEM((128,), jnp.int32),
            pltpu.VMEM((128,), jnp.int32),
            pltpu.VMEM((SEGS_PER_TILE * D_EMB,), jnp.float32),
            pltpu.VMEM((SEGS_PER_TILE * D_EMB,), jnp.float32),
            pltpu.SemaphoreType.DMA,
        ],
    )(q128, bkt.reshape(NTILES, NTILES * BROW), pp.reshape(-1))
    xo = xo_f.reshape(N, D_EMB)

    out = pl.pallas_call(
        _k5_body,
        out_shape=jax.ShapeDtypeStruct((N, 32), jnp.float32),
    )(xo, W_fc, b_fc.reshape(1, -1))

    return out[None], lp.reshape(1, N, K_NN, 1)


# speedup vs baseline: 9.7596x; 1.0171x over previous
import jax
import jax.numpy as jnp
from jax import lax
from jax.experimental import pallas as pl
from jax.experimental.pallas import tpu as pltpu
from jax.experimental.pallas import tpu_sc as plsc

N = 4096
K_NN = 16
D_EMB = 64
DP = 128
NTILES = 32
SEGS_PER_TILE = N // NTILES
EDG = N * K_NN // 2
EPT = N * K_NN // NTILES


def _lanes(v):
    return jnp.full((16,), v, jnp.int32)


def _iota16():
    return lax.broadcasted_iota(jnp.int32, (16,), 0)


def _k1_body(x_ref, wpre_ref, bpre_ref, wdgm_ref, bdgm_ref, wconv_ref,
             bconv_ref, gx_ref, gx128_ref, sq_ref, pp_ref, q128_ref):
    pre = jnp.dot(x_ref[...], wpre_ref[...],
                  preferred_element_type=jnp.float32) + bpre_ref[...]
    xp = jnp.where(pre >= 0, pre, 0.1 * pre)
    gx = jnp.dot(xp, wdgm_ref[...],
                 preferred_element_type=jnp.float32) + bdgm_ref[...]
    gx_ref[...] = gx
    zpad = jnp.zeros((N, DP - D_EMB), jnp.float32)
    gx128_ref[...] = jnp.concatenate([gx, zpad], axis=1)
    sq_ref[...] = jnp.sum(gx * gx, axis=1, keepdims=True)
    wc = wconv_ref[...]
    w_top = wc[:D_EMB, :]
    w_bot = wc[D_EMB:, :]
    pp_ref[...] = jnp.dot(xp, w_top - w_bot,
                          preferred_element_type=jnp.float32) + bconv_ref[...]
    q = jnp.dot(xp, w_bot, preferred_element_type=jnp.float32)
    q128_ref[...] = jnp.concatenate([q, zpad], axis=1)


def _k2_body(gxr_ref, gx_ref, sqt_ref, sqr_ref, t_ref, idx_ref):
    d = (sqr_ref[...] + sqt_ref[...] - 2.0 * lax.dot_general(
        gxr_ref[...], gx_ref[...], (((1,), (1,)), ((), ())),
        preferred_element_type=jnp.float32)) * t_ref[0, 0]
    rows = d.shape[0]
    iota = lax.broadcasted_iota(jnp.int32, (rows, N), 1)
    cols = []
    for _ in range(K_NN):
        am = jnp.argmin(d, axis=1).astype(jnp.int32)
        cols.append(am)
        d = jnp.where(iota == am[:, None], jnp.inf, d)
    idx_ref[...] = jnp.stack(cols, axis=1)


def _sc_wid():
    return lax.axis_index("s") * 2 + lax.axis_index("c")


def _k4_body(idxt_hbm, gx128_hbm, g1_hbm, alist_v, arows_v, sem):
    w = _sc_wid()
    pltpu.sync_copy(idxt_hbm.at[pl.ds(w * EPT, EPT)], alist_v)

    def chunk(cc, _):
        pltpu.async_copy(gx128_hbm.at[alist_v.at[pl.ds(cc * 512, 512)]],
                         arows_v, sem).wait()
        pltpu.sync_copy(arows_v, g1_hbm.at[pl.ds(w * EPT + cc * 512, 512)])
        return 0

    lax.fori_loop(0, EPT // 512, chunk, 0)


def _k4b_body(g1_ref, gxr_ref, t_ref, lp_ref):
    diff = g1_ref[...][:, :, :D_EMB] - gxr_ref[...][:, None, :]
    lp_ref[...] = -t_ref[0, 0] * jnp.sum(diff * diff, axis=2)


BROW = 1152
EPS = EDG // NTILES


def _k3a_body(idxt_hbm, bkt_hbm, seg_st, src_st, bkt_v, cnt_v):
    w = _sc_wid()
    it = _iota16()
    pltpu.sync_copy(idxt_hbm.at[pl.ds(EDG + w * EPS, EPS)], seg_st)
    pltpu.sync_copy(idxt_hbm.at[pl.ds(w * EPS, EPS)], src_st)
    zz = jnp.zeros((16,), jnp.int32)

    def binit(i, _):
        bkt_v[pl.ds(i * 16, 16)] = zz
        return 0

    lax.fori_loop(0, NTILES * BROW // 16 + 1, binit, 0, unroll=8)
    cnt_v[pl.ds(0, 16)] = zz
    cnt_v[pl.ds(16, 16)] = zz
    cnt_v[pl.ds(32, 16)] = zz

    def scan_vec(i, _):
        sv = seg_st[pl.ds(i * 16, 16)]
        xv = src_st[pl.ds(i * 16, 16)]
        pk = jnp.bitwise_or(lax.shift_left(sv, 12), xv)
        ow = lax.shift_right_logical(sv, 7)
        for ji in range(16):
            o = ow[ji]
            cv = cnt_v[pl.ds(o, 16)]
            cnt = cv[0]
            addr = o * BROW + 1 + cnt
            base = addr & ~jnp.int32(15)
            lane = addr & jnp.int32(15)
            cur = bkt_v[pl.ds(base, 16)]
            bkt_v[pl.ds(base, 16)] = jnp.where(it == lane, pk[ji], cur)
            cnt_v[pl.ds(o, 16)] = jnp.where(it == 0, cnt + 1, cv)
        return 0

    lax.fori_loop(0, EPS // 16, scan_vec, 0)

    for o in range(NTILES):
        cv = cnt_v[pl.ds(o, 16)]
        head = bkt_v[pl.ds(o * BROW, 16)]
        bkt_v[pl.ds(o * BROW, 16)] = jnp.where(it == 0, cv[0], head)
        pltpu.sync_copy(bkt_v.at[pl.ds(o * BROW, BROW)],
                        bkt_hbm.at[o, w])


def _k3_body(q128_hbm, bkt_hbm, ppf_hbm, out_hbm, bkts_v, pk_own,
             m_v, qrows_v, qrows2_v, glist_v, glist2_v, slist_v, slist2_v,
             prows_v, sem, sem2):
    w = _sc_wid()
    lo = w * SEGS_PER_TILE
    it = _iota16()
    neg = jnp.full((16,), -jnp.inf, jnp.float32)

    def minit(i, _):
        m_v[pl.ds(i * 16, 16)] = neg
        return 0

    lax.fori_loop(0, SEGS_PER_TILE * D_EMB // 16, minit, 0, unroll=8)

    pltpu.sync_copy(bkt_hbm.at[w], bkts_v)
    total = jnp.int32(0)
    for s in range(NTILES):
        cnt_s = bkts_v[pl.ds(s * BROW, 16)][0]

        def copy_vec(kk, tot, s=s):
            pk_own[pl.ds(tot, 16)] = bkts_v[pl.ds(s * BROW + 1 + kk * 16,
                                                  16)]
            return tot + 16

        lax.fori_loop(0, lax.div(cnt_s + 15, 16), copy_vec, total)
        total = total + cnt_s

    zz = jnp.zeros((16,), jnp.int32)
    for j in range(8):
        pk_own[pl.ds(total + j * 16, 16)] = zz

    @pl.when(w >= 16)
    def _():
        q0 = (w - 16) * SEGS_PER_TILE
        pltpu.sync_copy(q128_hbm.at[pl.ds(q0, SEGS_PER_TILE)], qrows_v)

        def omax(jg, _2):
            for ji in range(16):
                s = jg * 16 + ji
                for c4 in range(4):
                    off = s * D_EMB + c4 * 16
                    m_v[pl.ds(off, 16)] = jnp.maximum(
                        m_v[pl.ds(off, 16)], qrows_v[s, pl.ds(c4 * 16, 16)])
            return 0

        lax.fori_loop(0, SEGS_PER_TILE // 16, omax, 0)

    nch = lax.div(total + 127, 128)
    gl = (glist_v, glist2_v)
    sl = (slist_v, slist2_v)
    qr = (qrows_v, qrows2_v)
    sm = (sem, sem2)

    def unpack(ci, b):
        base = ci * 128
        for k16 in range(8):
            pk = pk_own[pl.ds(base + k16 * 16, 16)]
            gl[b][pl.ds(k16 * 16, 16)] = jnp.bitwise_and(pk, 4095)
            sl[b][pl.ds(k16 * 16, 16)] = (
                lax.shift_right_logical(pk, 12) - _lanes(lo))

    def start(b):
        pltpu.async_copy(q128_hbm.at[gl[b]], qr[b], sm[b])

    def wait(b):
        pltpu.make_async_copy(q128_hbm.at[pl.ds(0, 128)], qr[b],
                              sm[b]).wait()

    def process(ci, b):
        base = ci * 128

        def erow(jg, _2, b=b):
            segv = sl[b][pl.ds(jg * 16, 16)]
            for ji in range(16):
                j = jg * 16 + ji

                @pl.when(base + j < total)
                def _3(j=j, ji=ji, segv=segv, b=b):
                    off = segv[ji] * D_EMB
                    for c4 in range(4):
                        m_v[pl.ds(off + c4 * 16, 16)] = jnp.maximum(
                            m_v[pl.ds(off + c4 * 16, 16)],
                            qr[b][j, pl.ds(c4 * 16, 16)])
            return 0

        lax.fori_loop(0, 8, erow, 0)

    @pl.when(nch > 0)
    def _p0():
        unpack(0, 0)
        start(0)

    def pairloop(cp, _):
        e0 = cp * 2
        e1 = cp * 2 + 1

        @pl.when(e1 < nch)
        def _a():
            unpack(e1, 1)
            start(1)

        wait(0)
        process(e0, 0)

        @pl.when(e1 + 1 < nch)
        def _b():
            unpack(e1 + 1, 0)
            start(0)

        @pl.when(e1 < nch)
        def _c():
            wait(1)
            process(e1, 1)

        return 0

    lax.fori_loop(0, lax.div(nch + 1, 2), pairloop, 0)

    pltpu.sync_copy(ppf_hbm.at[pl.ds(lo * D_EMB, SEGS_PER_TILE * D_EMB)],
                    prows_v)

    def fin(i, _):
        prows_v[pl.ds(i * 16, 16)] = jnp.maximum(
            prows_v[pl.ds(i * 16, 16)] + m_v[pl.ds(i * 16, 16)], 0.0)
        return 0

    lax.fori_loop(0, SEGS_PER_TILE * D_EMB // 16, fin, 0, unroll=8)
    pltpu.sync_copy(prows_v, out_hbm.at[pl.ds(lo * D_EMB,
                                              SEGS_PER_TILE * D_EMB)])


def _k5_body(xo_ref, wfc_ref, bfc_ref, out_ref):
    out_ref[...] = jnp.dot(xo_ref[...], wfc_ref[...],
                           preferred_element_type=jnp.float32) + bfc_ref[...]


def kernel(x, W_pre, b_pre, temperature, W_dgm, b_dgm, W_conv, b_conv, W_fc,
           b_fc):
    x2 = x[0]

    gx, gx128, sq, pp, q128 = pl.pallas_call(
        _k1_body,
        out_shape=[
            jax.ShapeDtypeStruct((N, D_EMB), jnp.float32),
            jax.ShapeDtypeStruct((N, DP), jnp.float32),
            jax.ShapeDtypeStruct((N, 1), jnp.float32),
            jax.ShapeDtypeStruct((N, D_EMB), jnp.float32),
            jax.ShapeDtypeStruct((N, DP), jnp.float32),
        ],
    )(x2, W_pre, b_pre.reshape(1, -1), W_dgm, b_dgm.reshape(1, -1), W_conv,
      b_conv.reshape(1, -1))

    t = jnp.exp(jnp.clip(temperature, -5.0, 5.0))
    sqt = sq.reshape(1, N)
    rows = 256
    idx = pl.pallas_call(
        _k2_body,
        grid=(N // rows,),
        in_specs=[
            pl.BlockSpec((rows, D_EMB), lambda i: (i, 0)),
            pl.BlockSpec((N, D_EMB), lambda i: (0, 0)),
            pl.BlockSpec((1, N), lambda i: (0, 0)),
            pl.BlockSpec((rows, 1), lambda i: (i, 0)),
            pl.BlockSpec((1, 1), lambda i: (0, 0)),
        ],
        out_specs=pl.BlockSpec((rows, K_NN), lambda i: (i, 0)),
        out_shape=jax.ShapeDtypeStruct((N, K_NN), jnp.int32),
    )(gx, gx, sqt, sq, t.reshape(1, 1))

    mesh = plsc.VectorSubcoreMesh(core_axis_name="c", subcore_axis_name="s")
    idx_t = idx.T.reshape(-1)

    g1 = pl.kernel(
        _k4_body,
        out_type=jax.ShapeDtypeStruct((N * K_NN, DP), jnp.float32),
        mesh=mesh,
        scratch_types=[
            pltpu.VMEM((EPT,), jnp.int32),
            pltpu.VMEM((512, DP), jnp.float32),
            pltpu.SemaphoreType.DMA,
        ],
    )(idx_t, gx128)

    lp = pl.pallas_call(
        _k4b_body,
        grid=(8,),
        in_specs=[
            pl.BlockSpec((512, K_NN, DP), lambda i: (i, 0, 0)),
            pl.BlockSpec((512, D_EMB), lambda i: (i, 0)),
            pl.BlockSpec((1, 1), lambda i: (0, 0)),
        ],
        out_specs=pl.BlockSpec((512, K_NN), lambda i: (i, 0)),
        out_shape=jax.ShapeDtypeStruct((N, K_NN), jnp.float32),
    )(g1.reshape(N, K_NN, DP), gx, t.reshape(1, 1))

    bkt = pl.kernel(
        _k3a_body,
        out_type=jax.ShapeDtypeStruct((NTILES, NTILES, BROW), jnp.int32),
        mesh=mesh,
        scratch_types=[
            pltpu.VMEM((EPS,), jnp.int32),
            pltpu.VMEM((EPS,), jnp.int32),
            pltpu.VMEM((NTILES * BROW + 16,), jnp.int32),
            pltpu.VMEM((48,), jnp.int32),
        ],
    )(idx_t)

    xo_f = pl.kernel(
        _k3_body,
        out_type=jax.ShapeDtypeStruct((N * D_EMB,), jnp.float32),
        mesh=mesh,
        scratch_types=[
            pltpu.VMEM((NTILES * BROW,), jnp.int32),
            pltpu.VMEM((EDG + 256,), jnp.int32),
            pltpu.VMEM((SEGS_PER_TILE * D_EMB,), jnp.float32),
            pltpu.VMEM((128, DP), jnp.float32),
            pltpu.VMEM((128, DP), jnp.float32),
            pltpu.VMEM((128,), jnp.int32),
            pltpu.VMEM((128,), jnp.int32),
            pltpu.VMEM((128,), jnp.int32),
            pltpu.VMEM((128,), jnp.int32),
            pltpu.VMEM((SEGS_PER_TILE * D_EMB,), jnp.float32),
            pltpu.SemaphoreType.DMA,
            pltpu.SemaphoreType.DMA,
        ],
    )(q128, bkt.reshape(NTILES, NTILES * BROW), pp.reshape(-1))
    xo = xo_f.reshape(N, D_EMB)

    out = pl.pallas_call(
        _k5_body,
        out_shape=jax.ShapeDtypeStruct((N, 32), jnp.float32),
    )(xo, W_fc, b_fc.reshape(1, -1))

    return out[None], lp.reshape(1, N, K_NN, 1)
